# baseline (speedup 1.0000x reference)
import functools

import jax
import jax.numpy as jnp
from jax import lax
from jax.experimental import pallas as pl
from jax.experimental.pallas import tpu as pltpu
from jax.experimental.pallas import tpu_sc as plsc

B = 16384
E = 128
D = 128
NC = 2
NS = 16
NW = NC * NS
K = 2
CB = B // K
ROWS_PER_W = CB // NW
CHUNK = 128
NCHUNK = ROWS_PER_W // CHUNK
BB = 4096


def _sc_gather_chunk(label, table, c):
    mesh = plsc.VectorSubcoreMesh(core_axis_name="c", subcore_axis_name="s")

    @functools.partial(
        pl.kernel,
        mesh=mesh,
        out_type=jax.ShapeDtypeStruct((CB, E), jnp.float32),
        scratch_types=[
            pltpu.VMEM((ROWS_PER_W,), jnp.int32),
            pltpu.VMEM((ROWS_PER_W, E), jnp.float32),
            pltpu.SemaphoreType.DMA,
        ],
    )
    def gather_kernel(label_hbm, table_hbm, out_hbm, idx_v, rows_v, sem):
        wid = lax.axis_index("s") * NC + lax.axis_index("c")
        base = wid * ROWS_PER_W
        pltpu.sync_copy(label_hbm.at[pl.ds(c * CB + base, ROWS_PER_W)], idx_v)
        copies = [
            pltpu.async_copy(
                table_hbm.at[idx_v.at[pl.ds(j * CHUNK, CHUNK)]],
                rows_v.at[pl.ds(j * CHUNK, CHUNK)],
                sem,
            )
            for j in range(NCHUNK)
        ]
        for cp in copies:
            cp.wait()
        pltpu.sync_copy(rows_v, out_hbm.at[pl.ds(base, ROWS_PER_W)])

    return gather_kernel(label, table)


def _tc_chunk(x, W, b2, c, y_prev=None):
    nblk = CB // BB

    def body(*refs):
        x_ref, w_ref, b_ref = refs[0], refs[1], refs[2]
        o_ref = refs[-1]
        y = jnp.dot(x_ref[...], w_ref[...], preferred_element_type=jnp.float32)
        y = y + b_ref[...]
        o_ref[...] = (y * jax.nn.sigmoid(y))[:, None, :]

    in_specs = [
        pl.BlockSpec((BB, E), lambda j: (j, 0)),
        pl.BlockSpec((E, D), lambda j: (0, 0)),
        pl.BlockSpec((1, D), lambda j: (0, 0)),
    ]
    args = [x, W, b2]
    aliases = {}
    if y_prev is not None:
        in_specs.append(pl.BlockSpec(memory_space=pl.ANY))
        args.append(y_prev)
        aliases = {3: 0}
    return pl.pallas_call(
        body,
        grid=(nblk,),
        in_specs=in_specs,
        out_specs=pl.BlockSpec((BB, 1, D), lambda j: (c * nblk + j, 0, 0)),
        out_shape=jax.ShapeDtypeStruct((B, 1, D), jnp.float32),
        input_output_aliases=aliases,
    )(*args)


def kernel(label, table, W, b):
    b2 = b.reshape(1, D)
    x0 = _sc_gather_chunk(label, table, 0)
    x1 = _sc_gather_chunk(label, table, 1)
    y0 = _tc_chunk(x0, W, b2, 0)
    return _tc_chunk(x1, W, b2, 1, y_prev=y0)
